# baseline (speedup 1.0000x reference)
import functools

import jax
import jax.numpy as jnp
from jax import lax
from jax.experimental import pallas as pl
from jax.experimental.pallas import tpu as pltpu
from jax.experimental.pallas import tpu_sc as plsc

_NC = 2
_NS = 16
_L = 16
_NW = _NC * _NS


@functools.lru_cache(maxsize=None)
def _build_tc_add(nb, S, D, SB):
    def body(x_ref, pe_ref, o_ref):
        o_ref[...] = x_ref[...] + pe_ref[...][None]

    return pl.pallas_call(
        body,
        grid=(S // SB, nb),
        in_specs=[
            pl.BlockSpec((1, SB, D), lambda s, b: (b, s, 0)),
            pl.BlockSpec((SB, D), lambda s, b: (s, 0)),
        ],
        out_specs=pl.BlockSpec((1, SB, D), lambda s, b: (b, s, 0)),
        out_shape=jax.ShapeDtypeStruct((nb, S, D), jnp.float32),
    )


@functools.lru_cache(maxsize=None)
def _build_sc_add(B, S, D, CH, b0=0, nb=None):
    if nb is None:
        nb = B
    seq_per_w = S // _NW
    n_chunks = seq_per_w // CH
    n_col = D // _L
    G = 8
    mesh = plsc.VectorSubcoreMesh(
        core_axis_name="c", subcore_axis_name="s",
        num_cores=_NC, num_subcores=_NS)

    @functools.partial(
        pl.kernel,
        out_type=jax.ShapeDtypeStruct((nb, S, D), jnp.float32),
        mesh=mesh,
        scratch_types=[
            pltpu.VMEM((3, nb, CH, D), jnp.float32),
            pltpu.VMEM((2, CH, D), jnp.float32),
            pltpu.SemaphoreType.DMA((3, nb)),
            pltpu.SemaphoreType.DMA((3, nb)),
            pltpu.SemaphoreType.DMA((2,)),
        ],
    )
    def run(x_hbm, pe_hbm, out_hbm, x_sl, pe_sl, in_sems, out_sems,
            pe_sems):
        wid = lax.axis_index("s") * _NC + lax.axis_index("c")
        s_base = wid * seq_per_w

        def start_pe(t, p):
            pltpu.async_copy(pe_hbm.at[pl.ds(s_base + t * CH, CH)],
                             pe_sl.at[p], pe_sems.at[p])

        def wait_pe(t, p):
            pltpu.make_async_copy(pe_hbm.at[pl.ds(s_base + t * CH, CH)],
                                  pe_sl.at[p], pe_sems.at[p]).wait()

        def start_in(t, p, b):
            pltpu.async_copy(x_hbm.at[b0 + b, pl.ds(s_base + t * CH, CH)],
                             x_sl.at[p, b], in_sems.at[p, b])

        def wait_in(t, p, b):
            pltpu.make_async_copy(
                x_hbm.at[b0 + b, pl.ds(s_base + t * CH, CH)],
                                  x_sl.at[p, b], in_sems.at[p, b]).wait()

        def start_out(t, p, b):
            pltpu.async_copy(x_sl.at[p, b],
                             out_hbm.at[b, pl.ds(s_base + t * CH, CH)],
                             out_sems.at[p, b])

        def wait_out(t, p, b):
            pltpu.make_async_copy(x_sl.at[p, b],
                                  out_hbm.at[b, pl.ds(s_base + t * CH, CH)],
                                  out_sems.at[p, b]).wait()

        start_pe(0, 0)
        for b in range(nb):
            start_in(0, 0, b)
        start_pe(1, 1)
        for b in range(nb):
            start_in(1, 1, b)

        def body(t, carry):
            p = lax.rem(t, 3)
            pp = lax.rem(t, 2)
            q = lax.rem(t + 2, 3)
            wait_pe(t, pp)
            for b in range(nb):
                wait_in(t, p, b)

            @plsc.parallel_loop(0, CH)
            def _(r):
                for g in range(n_col // G):
                    cols = [(g * G + j) * _L for j in range(G)]
                    pe_vs = [pe_sl[pp, r, pl.ds(c, _L)] for c in cols]
                    for b in range(nb):
                        xs = [x_sl[p, b, r, pl.ds(c, _L)] for c in cols]
                        for c, xv, pv in zip(cols, xs, pe_vs):
                            x_sl[p, b, r, pl.ds(c, _L)] = xv + pv

            @pl.when(t < n_chunks - 2)
            def _():
                for b in range(nb):
                    @pl.when(t > 0)
                    def _():
                        wait_out(t - 1, q, b)
                    start_in(t + 2, q, b)

            for b in range(nb):
                start_out(t, p, b)

            @pl.when(t < n_chunks - 2)
            def _():
                start_pe(t + 2, pp)
            return carry

        lax.fori_loop(0, n_chunks, body, 0)

        for t in range(n_chunks - 3, n_chunks):
            for b in range(nb):
                wait_out(t, t % 3, b)

    return run


def kernel(x, pe):
    B, S, D = x.shape
    nb_tc = B // 2
    tc = _build_tc_add(nb_tc, S, D, 512)
    sc = _build_sc_add(B, S, D, 8, b0=nb_tc, nb=B - nb_tc)
    out_tc = tc(x, pe)
    out_sc = sc(x, pe)
    return jnp.concatenate([out_tc, out_sc], axis=0)


# speedup vs baseline: 1.5573x; 1.5573x over previous
import functools

import jax
import jax.numpy as jnp
from jax import lax
from jax.experimental import pallas as pl
from jax.experimental.pallas import tpu as pltpu
from jax.experimental.pallas import tpu_sc as plsc

_NC = 2
_NS = 16
_L = 16
_NW = _NC * _NS


@functools.lru_cache(maxsize=None)
def _build_sc_add(B, S, D, CH):
    seq_per_w = S // _NW
    n_chunks = seq_per_w // CH
    n_col = D // _L
    G = 8
    mesh = plsc.VectorSubcoreMesh(
        core_axis_name="c", subcore_axis_name="s",
        num_cores=_NC, num_subcores=_NS)

    @functools.partial(
        pl.kernel,
        out_type=jax.ShapeDtypeStruct((B, S, D), jnp.float32),
        mesh=mesh,
        scratch_types=[
            pltpu.VMEM((3, B, CH, D), jnp.float32),
            pltpu.VMEM((2, CH, D), jnp.float32),
            pltpu.SemaphoreType.DMA((3,)),
            pltpu.SemaphoreType.DMA((3,)),
            pltpu.SemaphoreType.DMA((2,)),
        ],
    )
    def run(x_hbm, pe_hbm, out_hbm, x_sl, pe_sl, in_sems, out_sems,
            pe_sems):
        wid = lax.axis_index("s") * _NC + lax.axis_index("c")
        s_base = wid * seq_per_w

        def start_pe(t, p):
            pltpu.async_copy(pe_hbm.at[pl.ds(s_base + t * CH, CH)],
                             pe_sl.at[p], pe_sems.at[p])

        def wait_pe(t, p):
            pltpu.make_async_copy(pe_hbm.at[pl.ds(s_base + t * CH, CH)],
                                  pe_sl.at[p], pe_sems.at[p]).wait()

        def start_in(t, p):
            pltpu.async_copy(x_hbm.at[:, pl.ds(s_base + t * CH, CH)],
                             x_sl.at[p], in_sems.at[p])

        def wait_in(t, p):
            pltpu.make_async_copy(x_hbm.at[:, pl.ds(s_base + t * CH, CH)],
                                  x_sl.at[p], in_sems.at[p]).wait()

        def start_out(t, p):
            pltpu.async_copy(x_sl.at[p],
                             out_hbm.at[:, pl.ds(s_base + t * CH, CH)],
                             out_sems.at[p])

        def wait_out(t, p):
            pltpu.make_async_copy(x_sl.at[p],
                                  out_hbm.at[:, pl.ds(s_base + t * CH, CH)],
                                  out_sems.at[p]).wait()

        start_pe(0, 0)
        start_in(0, 0)
        start_pe(1, 1)
        start_in(1, 1)

        def body(t, carry):
            p = lax.rem(t, 3)
            pp = lax.rem(t, 2)
            q = lax.rem(t + 2, 3)
            wait_pe(t, pp)
            wait_in(t, p)

            @plsc.parallel_loop(0, CH)
            def _(r):
                for g in range(n_col // G):
                    cols = [(g * G + j) * _L for j in range(G)]
                    pe_vs = [pe_sl[pp, r, pl.ds(c, _L)] for c in cols]
                    for b in range(B):
                        xs = [x_sl[p, b, r, pl.ds(c, _L)] for c in cols]
                        for c, xv, pv in zip(cols, xs, pe_vs):
                            x_sl[p, b, r, pl.ds(c, _L)] = xv + pv

            @pl.when(t < n_chunks - 2)
            def _():
                @pl.when(t > 0)
                def _():
                    wait_out(t - 1, q)
                start_in(t + 2, q)

            start_out(t, p)

            @pl.when(t < n_chunks - 2)
            def _():
                start_pe(t + 2, pp)
            return carry

        lax.fori_loop(0, n_chunks, body, 0)

        for t in range(n_chunks - 3, n_chunks):
            wait_out(t, t % 3)

    return run


def kernel(x, pe):
    B, S, D = x.shape
    run = _build_sc_add(B, S, D, 8)
    return run(x, pe)


# speedup vs baseline: 1.5649x; 1.0049x over previous
import functools

import jax
import jax.numpy as jnp
from jax import lax
from jax.experimental import pallas as pl
from jax.experimental.pallas import tpu as pltpu
from jax.experimental.pallas import tpu_sc as plsc

_NC = 2
_NS = 16
_L = 16
_NW = _NC * _NS


@functools.lru_cache(maxsize=None)
def _build_sc_add(B, S, D, CH):
    seq_per_w = S // _NW
    n_chunks = seq_per_w // CH
    n_col = D // _L
    G = 8
    mesh = plsc.VectorSubcoreMesh(
        core_axis_name="c", subcore_axis_name="s",
        num_cores=_NC, num_subcores=_NS)

    @functools.partial(
        pl.kernel,
        out_type=jax.ShapeDtypeStruct((B, S, D), jnp.float32),
        mesh=mesh,
        scratch_types=[
            pltpu.VMEM((3, B, CH, D), jnp.float32),
            pltpu.VMEM((2, CH, D), jnp.float32),
            pltpu.SemaphoreType.DMA((3,)),
            pltpu.SemaphoreType.DMA((3,)),
            pltpu.SemaphoreType.DMA((2,)),
        ],
    )
    def run(x_hbm, pe_hbm, out_hbm, x_sl, pe_sl, in_sems, out_sems,
            pe_sems):
        wid = lax.axis_index("s") * _NC + lax.axis_index("c")
        s_base = wid * seq_per_w

        def start_pe(t, p):
            pltpu.async_copy(pe_hbm.at[pl.ds(s_base + t * CH, CH)],
                             pe_sl.at[p], pe_sems.at[p])

        def wait_pe(t, p):
            pltpu.make_async_copy(pe_hbm.at[pl.ds(s_base + t * CH, CH)],
                                  pe_sl.at[p], pe_sems.at[p]).wait()

        def start_in(t, p):
            pltpu.async_copy(x_hbm.at[:, pl.ds(s_base + t * CH, CH)],
                             x_sl.at[p], in_sems.at[p])

        def wait_in(t, p):
            pltpu.make_async_copy(x_hbm.at[:, pl.ds(s_base + t * CH, CH)],
                                  x_sl.at[p], in_sems.at[p]).wait()

        def start_out(t, p):
            pltpu.async_copy(x_sl.at[p],
                             out_hbm.at[:, pl.ds(s_base + t * CH, CH)],
                             out_sems.at[p])

        def wait_out(t, p):
            pltpu.make_async_copy(x_sl.at[p],
                                  out_hbm.at[:, pl.ds(s_base + t * CH, CH)],
                                  out_sems.at[p]).wait()

        start_pe(0, 0)
        start_in(0, 0)
        start_pe(1, 1)
        start_in(1, 1)

        def body(t, carry):
            p = lax.rem(t, 3)
            pp = lax.rem(t, 2)
            q = lax.rem(t + 2, 3)
            wait_pe(t, pp)
            wait_in(t, p)

            @plsc.parallel_loop(0, CH)
            def _(r):
                for g in range(n_col // G):
                    cols = [(g * G + j) * _L for j in range(G)]
                    pe_vs = [pe_sl[pp, r, pl.ds(c, _L)] for c in cols]
                    for b in range(B):
                        for c, pv in zip(cols, pe_vs):
                            plsc.addupdate(x_sl.at[p, b, r, pl.ds(c, _L)],
                                           pv)

            @pl.when(t < n_chunks - 2)
            def _():
                @pl.when(t > 0)
                def _():
                    wait_out(t - 1, q)
                start_in(t + 2, q)

            start_out(t, p)

            @pl.when(t < n_chunks - 2)
            def _():
                start_pe(t + 2, pp)
            return carry

        lax.fori_loop(0, n_chunks, body, 0)

        for t in range(n_chunks - 3, n_chunks):
            wait_out(t, t % 3)

    return run


def kernel(x, pe):
    B, S, D = x.shape
    run = _build_sc_add(B, S, D, 8)
    return run(x, pe)
